# baseline (speedup 1.0000x reference)
import functools

import jax
import jax.numpy as jnp
from jax import lax
from jax.experimental import pallas as pl
from jax.experimental.pallas import tpu as pltpu
from jax.experimental.pallas import tpu_sc as plsc

B, T, D = 16, 1024, 32
K = 8192
BT = B * T

M_TILE = 128
K_TILE = K
M_TILES = BT // M_TILE
K_TILES = K // K_TILE

SC_CORES = 2
SC_SUBCORES = 16
SC_WORKERS = SC_CORES * SC_SUBCORES
ROWS_PER_WORKER = BT // SC_WORKERS
IDX_CHUNK = 128
CHUNKS_PER_WORKER = ROWS_PER_WORKER // IDX_CHUNK


def _vq_body(z_ref, cb_ref, sim_ref, e_ref, hcb_ref, cbs_ref):
    m = pl.program_id(0)

    @pl.when(m == 0)
    def _():
        cb = cb_ref[...]
        norm_col = jnp.sum(cb * cb, axis=1, keepdims=True)
        cbs_ref[...] = cb * lax.rsqrt(norm_col)
        norm_row = jnp.sum(cb * cb, axis=1)[None, :]
        hcb_ref[...] = jnp.broadcast_to(0.5 * norm_row, (8, K))

    z = z_ref[...]
    norm_z = jnp.sum(z * z, axis=1, keepdims=True)
    zs = z * lax.rsqrt(norm_z)
    sim_ref[...] = lax.dot_general(zs, cbs_ref[...], (((1,), (1,)), ((), ())),
                                   preferred_element_type=jnp.float32)

    dot = lax.dot_general(z, cb_ref[...], (((1,), (1,)), ((), ())),
                          preferred_element_type=jnp.float32)
    dist = (hcb_ref[...][None] - dot.reshape(M_TILE // 8, 8, K))
    dist = dist.reshape(M_TILE, K)
    e_ref[...] = jnp.argmin(dist, axis=1).astype(jnp.int32)[:, None]


_vq_call = pl.pallas_call(
    _vq_body,
    grid=(M_TILES,),
    in_specs=[
        pl.BlockSpec((M_TILE, D), lambda m: (m, 0)),
        pl.BlockSpec((K, D), lambda m: (0, 0)),
    ],
    out_specs=[
        pl.BlockSpec((M_TILE, K), lambda m: (m, 0)),
        pl.BlockSpec((M_TILE, 1), lambda m: (m, 0)),
    ],
    out_shape=[
        jax.ShapeDtypeStruct((BT, K), jnp.float32),
        jax.ShapeDtypeStruct((BT, 1), jnp.int32),
    ],
    scratch_shapes=[
        pltpu.VMEM((8, K), jnp.float32),
        pltpu.VMEM((K, D), jnp.float32),
    ],
    compiler_params=pltpu.CompilerParams(
        dimension_semantics=("arbitrary",),
    ),
)


@functools.partial(
    pl.kernel,
    mesh=plsc.VectorSubcoreMesh(core_axis_name="c", subcore_axis_name="s"),
    compiler_params=pltpu.CompilerParams(use_tc_tiling_on_sc=False),
    out_type=jax.ShapeDtypeStruct((BT, D), jnp.float32),
    scratch_types=[
        pltpu.VMEM((CHUNKS_PER_WORKER, IDX_CHUNK), jnp.int32),
        pltpu.VMEM((ROWS_PER_WORKER, D), jnp.float32),
        pltpu.SemaphoreType.DMA,
    ],
)
def _gather_rows(cb_hbm, idx_hbm, out_hbm, idx_v, rows_v, sem):
    wid = lax.axis_index("s") * SC_CORES + lax.axis_index("c")
    base = wid * ROWS_PER_WORKER
    pltpu.sync_copy(idx_hbm.at[pl.ds(wid * CHUNKS_PER_WORKER, CHUNKS_PER_WORKER)],
                    idx_v)
    copies = []
    for j in range(CHUNKS_PER_WORKER):
        copies.append(pltpu.async_copy(
            cb_hbm.at[idx_v.at[j]],
            rows_v.at[pl.ds(j * IDX_CHUNK, IDX_CHUNK)],
            sem))
    for c in copies:
        c.wait()
    pltpu.sync_copy(rows_v, out_hbm.at[pl.ds(base, ROWS_PER_WORKER)])


def kernel(z, codebook):
    zf = z.reshape(BT, D)
    sim, e2d = _vq_call(zf, codebook)
    idx = e2d.reshape(BT // IDX_CHUNK, IDX_CHUNK)
    z_q = _gather_rows(codebook, idx)
    return (e2d.reshape(B, T),
            z_q.reshape(B, T, D),
            sim.reshape(B, T, K))


# speedup vs baseline: 1.1480x; 1.1480x over previous
import functools

import jax
import jax.numpy as jnp
from jax import lax
from jax.experimental import pallas as pl
from jax.experimental.pallas import tpu as pltpu
from jax.experimental.pallas import tpu_sc as plsc

B, T, D = 16, 1024, 32
K = 8192
BT = B * T

M_TILE = 512
K_TILE = K
M_TILES = BT // M_TILE
K_TILES = K // K_TILE

SC_CORES = 2
SC_SUBCORES = 16
SC_WORKERS = SC_CORES * SC_SUBCORES
ROWS_PER_WORKER = BT // SC_WORKERS
IDX_CHUNK = 128
CHUNKS_PER_WORKER = ROWS_PER_WORKER // IDX_CHUNK


def _vq_body(z_ref, cb_ref, sim_ref, e_ref, hcb_ref, cbs_ref):
    m = pl.program_id(0)

    @pl.when(m == 0)
    def _():
        cb = cb_ref[...]
        norm_col = jnp.sum(cb * cb, axis=1, keepdims=True)
        cbs_ref[...] = cb * lax.rsqrt(norm_col)
        norm_row = jnp.sum(cb * cb, axis=1)[None, :]
        hcb_ref[...] = jnp.broadcast_to(0.5 * norm_row, (8, K))

    z = z_ref[...]
    norm_z = jnp.sum(z * z, axis=1, keepdims=True)
    zs = z * lax.rsqrt(norm_z)
    sim_ref[...] = lax.dot_general(zs, cbs_ref[...], (((1,), (1,)), ((), ())),
                                   preferred_element_type=jnp.float32)

    dot = lax.dot_general(z, cb_ref[...], (((1,), (1,)), ((), ())),
                          preferred_element_type=jnp.float32)
    dist = (hcb_ref[...][None] - dot.reshape(M_TILE // 8, 8, K))
    dist = dist.reshape(M_TILE, K)
    e_ref[...] = jnp.argmin(dist, axis=1).astype(jnp.int32)[:, None]


_vq_call = pl.pallas_call(
    _vq_body,
    grid=(M_TILES,),
    in_specs=[
        pl.BlockSpec((M_TILE, D), lambda m: (m, 0)),
        pl.BlockSpec((K, D), lambda m: (0, 0)),
    ],
    out_specs=[
        pl.BlockSpec((M_TILE, K), lambda m: (m, 0)),
        pl.BlockSpec((M_TILE, 1), lambda m: (m, 0)),
    ],
    out_shape=[
        jax.ShapeDtypeStruct((BT, K), jnp.float32),
        jax.ShapeDtypeStruct((BT, 1), jnp.int32),
    ],
    scratch_shapes=[
        pltpu.VMEM((8, K), jnp.float32),
        pltpu.VMEM((K, D), jnp.float32),
    ],
    compiler_params=pltpu.CompilerParams(
        dimension_semantics=("arbitrary",),
    ),
)


@functools.partial(
    pl.kernel,
    mesh=plsc.VectorSubcoreMesh(core_axis_name="c", subcore_axis_name="s"),
    compiler_params=pltpu.CompilerParams(use_tc_tiling_on_sc=False),
    out_type=jax.ShapeDtypeStruct((BT, D), jnp.float32),
    scratch_types=[
        pltpu.VMEM((CHUNKS_PER_WORKER, IDX_CHUNK), jnp.int32),
        pltpu.VMEM((ROWS_PER_WORKER, D), jnp.float32),
        pltpu.SemaphoreType.DMA,
    ],
)
def _gather_rows(cb_hbm, idx_hbm, out_hbm, idx_v, rows_v, sem):
    wid = lax.axis_index("s") * SC_CORES + lax.axis_index("c")
    base = wid * ROWS_PER_WORKER
    pltpu.sync_copy(idx_hbm.at[pl.ds(wid * CHUNKS_PER_WORKER, CHUNKS_PER_WORKER)],
                    idx_v)
    copies = []
    for j in range(CHUNKS_PER_WORKER):
        copies.append(pltpu.async_copy(
            cb_hbm.at[idx_v.at[j]],
            rows_v.at[pl.ds(j * IDX_CHUNK, IDX_CHUNK)],
            sem))
    for c in copies:
        c.wait()
    pltpu.sync_copy(rows_v, out_hbm.at[pl.ds(base, ROWS_PER_WORKER)])


def kernel(z, codebook):
    zf = z.reshape(BT, D)
    sim, e2d = _vq_call(zf, codebook)
    idx = e2d.reshape(BT // IDX_CHUNK, IDX_CHUNK)
    z_q = _gather_rows(codebook, idx)
    return (e2d.reshape(B, T),
            z_q.reshape(B, T, D),
            sim.reshape(B, T, K))
